# baseline (speedup 1.0000x reference)
import jax
import jax.numpy as jnp
from jax import lax
from jax.experimental import pallas as pl

_DN_T = (((1,), (1,)), ((), ()))


def _moe_body(H, x_ref, Wg_ref, bg_ref, Wef_ref, bef_ref, Wc_ref, bc_ref,
              logits_ref, sparse_ref, mixed_ref, full_ref):
    E = Wg_ref.shape[0]
    BT = x_ref.shape[0]
    xt = x_ref[...]

    gl = lax.dot_general(xt, Wg_ref[...], _DN_T,
                         preferred_element_type=jnp.float32)
    gl = gl + bg_ref[...][None, :]
    m = jnp.max(gl, axis=1, keepdims=True)
    eg = jnp.exp(gl - m)
    probs = eg / jnp.sum(eg, axis=1, keepdims=True)
    full_ref[...] = probs

    e_iota = lax.broadcasted_iota(jnp.int32, (BT, E), 1)
    v1 = jnp.max(probs, axis=1, keepdims=True)
    i1 = jnp.min(jnp.where(probs == v1, e_iota, E), axis=1, keepdims=True)
    probs2 = jnp.where(e_iota == i1, -1.0, probs)
    v2 = jnp.max(probs2, axis=1, keepdims=True)
    i2 = jnp.min(jnp.where(probs2 == v2, e_iota, E), axis=1, keepdims=True)
    s = v1 + v2
    sparse = jnp.where(e_iota == i1, v1 / s, 0.0) + jnp.where(e_iota == i2, v2 / s, 0.0)
    sparse_ref[...] = sparse

    z_all = lax.dot_general(xt, Wef_ref[...], _DN_T,
                            preferred_element_type=jnp.float32)
    h_all = jnp.maximum(z_all + bef_ref[...][None, :], 0.0)
    acc = sparse[:, 0:1] * h_all[:, 0:H]
    for e in range(1, E):
        acc = acc + sparse[:, e:e + 1] * h_all[:, e * H:(e + 1) * H]
    mixed_ref[...] = acc

    logits_ref[...] = (
        lax.dot_general(acc, Wc_ref[...], _DN_T,
                        preferred_element_type=jnp.float32)
        + bc_ref[...][None, :]
    )


def kernel(x, Wg, bg, We, be, Wc, bc):
    import functools
    B, D = x.shape
    E, H, _ = We.shape
    C = Wc.shape[0]

    Wef = We.reshape(E * H, D)
    bef = be.reshape(E * H)

    BT = 512 if B % 512 == 0 else B
    grid = (B // BT,)

    logits, sparse, mixed, full = pl.pallas_call(
        functools.partial(_moe_body, H),
        grid=grid,
        in_specs=[
            pl.BlockSpec((BT, D), lambda i: (i, 0)),
            pl.BlockSpec((E, D), lambda i: (0, 0)),
            pl.BlockSpec((E,), lambda i: (0,)),
            pl.BlockSpec((E * H, D), lambda i: (0, 0)),
            pl.BlockSpec((E * H,), lambda i: (0,)),
            pl.BlockSpec((C, H), lambda i: (0, 0)),
            pl.BlockSpec((C,), lambda i: (0,)),
        ],
        out_specs=[
            pl.BlockSpec((BT, C), lambda i: (i, 0)),
            pl.BlockSpec((BT, E), lambda i: (i, 0)),
            pl.BlockSpec((BT, H), lambda i: (i, 0)),
            pl.BlockSpec((BT, E), lambda i: (i, 0)),
        ],
        out_shape=[
            jax.ShapeDtypeStruct((B, C), jnp.float32),
            jax.ShapeDtypeStruct((B, E), jnp.float32),
            jax.ShapeDtypeStruct((B, H), jnp.float32),
            jax.ShapeDtypeStruct((B, E), jnp.float32),
        ],
    )(x, Wg, bg, Wef, bef, Wc, bc)

    return (logits, sparse, mixed, full)
